# baseline (speedup 1.0000x reference)
import dataclasses

import jax
import jax.numpy as jnp
from jax import lax
from jax.experimental import pallas as pl
from jax.experimental.pallas import tpu as pltpu
from jax.experimental.pallas import tpu_sc as plsc

N = 10000
E = 320000
D = 128

NC = 2
NS = 16
NW = NC * NS
LANES = 16

EPW = E // NW
B = 256
NHALF = 5000
NR = 5120
DUMP = NHALF
CH = NR // NS
K = 10240
MAXNB = K // B
NL = NW * NC

_vec_mesh = plsc.VectorSubcoreMesh(
    core_axis_name="c", subcore_axis_name="s", num_cores=NC, num_subcores=NS
)

_sc_params = pltpu.CompilerParams()
if "needs_layout_passes" in pltpu.CompilerParams.__dataclass_fields__:
  _sc_params = dataclasses.replace(_sc_params, needs_layout_passes=False)


def _sc_degrees(src3, dst3):
  @pl.kernel(
      out_type=(
          jax.ShapeDtypeStruct((NW, 1, N), jnp.float32),
          jax.ShapeDtypeStruct((NW, 1, N), jnp.float32),
          jax.ShapeDtypeStruct((NL, MAXNB, 1, B), jnp.int32),
          jax.ShapeDtypeStruct((NL, MAXNB, 1, B), jnp.int32),
          jax.ShapeDtypeStruct((NL, 1, 16), jnp.int32),
      ),
      mesh=_vec_mesh,
      scratch_types=[
          pltpu.VMEM((1, EPW), jnp.int32),
          pltpu.VMEM((1, EPW), jnp.int32),
          pltpu.VMEM((1, N), jnp.float32),
          pltpu.VMEM((1, N), jnp.float32),
          pltpu.VMEM((MAXNB, 1, B), jnp.int32),
          pltpu.VMEM((MAXNB, 1, B), jnp.int32),
          pltpu.VMEM((MAXNB, 1, B), jnp.int32),
          pltpu.VMEM((MAXNB, 1, B), jnp.int32),
          pltpu.VMEM((1, 16), jnp.int32),
      ],
      compiler_params=_sc_params,
  )
  def deg_kernel(src_hbm, dst_hbm, dout_hbm, din_hbm, srcl_hbm, dstl_hbm,
                 cnts_hbm, sv, dv, aout, ain, sl0, dl0, sl1, dl1, cb):
    cid = lax.axis_index("c")
    sid = lax.axis_index("s")
    wid = sid * NC + cid

    pltpu.sync_copy(src_hbm.at[wid], sv)
    pltpu.sync_copy(dst_hbm.at[wid], dv)

    zeros = jnp.zeros((LANES,), jnp.float32)
    ones = jnp.ones((LANES,), jnp.float32)
    izeros = jnp.zeros((LANES,), jnp.int32)
    iones = jnp.ones((LANES,), jnp.int32)
    idump = jnp.full((LANES,), DUMP, jnp.int32)

    @pl.loop(0, N, step=LANES)
    def _(i):
      aout[0, pl.ds(i, LANES)] = zeros
      ain[0, pl.ds(i, LANES)] = zeros

    @pl.loop(0, MAXNB)
    def _(j):
      @pl.loop(0, B, step=LANES)
      def _(l):
        sl0[j, 0, pl.ds(l, LANES)] = izeros
        dl0[j, 0, pl.ds(l, LANES)] = idump
        sl1[j, 0, pl.ds(l, LANES)] = izeros
        dl1[j, 0, pl.ds(l, LANES)] = idump

    def edge_body(i, carry):
      off0, off1 = carry
      s = sv[0, pl.ds(i, LANES)]
      d = dv[0, pl.ds(i, LANES)]
      plsc.addupdate_scatter(aout.at[0], [s], ones)
      plsc.addupdate_scatter(ain.at[0], [d], ones)

      izero_v = jnp.zeros((LANES,), jnp.int32)

      m0 = d < NHALF
      c0 = plsc.cumsum(iones, mask=m0)
      pos0 = off0 + c0 - 1
      j0 = jnp.right_shift(pos0, 8)
      l0 = jnp.bitwise_and(pos0, B - 1)
      plsc.store_scatter(sl0, [j0, izero_v, l0], s, mask=m0)
      plsc.store_scatter(dl0, [j0, izero_v, l0], d, mask=m0)
      n0 = plsc.all_reduce_population_count(m0)

      m1 = jnp.logical_not(m0)
      c1 = plsc.cumsum(iones, mask=m1)
      pos1 = off1 + c1 - 1
      j1 = jnp.right_shift(pos1, 8)
      l1 = jnp.bitwise_and(pos1, B - 1)
      plsc.store_scatter(sl1, [j1, izero_v, l1], s, mask=m1)
      plsc.store_scatter(dl1, [j1, izero_v, l1], d - NHALF, mask=m1)
      n1 = plsc.all_reduce_population_count(m1)
      return (off0 + n0.astype(jnp.int32), off1 + n1.astype(jnp.int32))

    off0, off1 = pl.loop(0, EPW, step=LANES,
                         init_carry=(izeros, izeros))(edge_body)
    pltpu.sync_copy(aout, dout_hbm.at[wid])
    pltpu.sync_copy(ain, din_hbm.at[wid])
    pltpu.sync_copy(sl0, srcl_hbm.at[2 * wid])
    pltpu.sync_copy(dl0, dstl_hbm.at[2 * wid])
    pltpu.sync_copy(sl1, srcl_hbm.at[2 * wid + 1])
    pltpu.sync_copy(dl1, dstl_hbm.at[2 * wid + 1])
    cb[0, pl.ds(0, LANES)] = off0
    pltpu.sync_copy(cb, cnts_hbm.at[2 * wid])
    cb[0, pl.ds(0, LANES)] = off1
    pltpu.sync_copy(cb, cnts_hbm.at[2 * wid + 1])

  return deg_kernel(src3, dst3)


def _sc_segsum_rows(y, srcl, dstl, cnts):
  @pl.kernel(
      out_type=jax.ShapeDtypeStruct((NC, NR, D), jnp.float32),
      mesh=_vec_mesh,
      scratch_types=[
          pltpu.VMEM((1, B), jnp.int32),
          pltpu.VMEM((1, B), jnp.int32),
          pltpu.VMEM((1, B), jnp.int32),
          pltpu.VMEM((1, B), jnp.int32),
          pltpu.VMEM((B, D), jnp.float32),
          pltpu.VMEM((B, D), jnp.float32),
          pltpu.VMEM_SHARED((NR, D), jnp.float32),
          pltpu.VMEM((1, 16), jnp.int32),
          pltpu.SemaphoreType.DMA,
          pltpu.SemaphoreType.DMA,
      ],
      compiler_params=_sc_params,
  )
  def seg_kernel(y_hbm, srcl_hbm, dstl_hbm, cnts_hbm, zero_hbm, out_hbm,
                 si0, di0, si1, di1, rows0, rows1, acc, cb, sem0, sem1):
    cid = lax.axis_index("c")
    sid = lax.axis_index("s")

    roff = pl.multiple_of(sid * CH, 8)
    pltpu.sync_copy(zero_hbm.at[pl.ds(roff, CH)], acc.at[pl.ds(roff, CH)])
    plsc.subcore_barrier()

    def g_start(si, rows, sem):
      pltpu.async_copy(y_hbm.at[si.at[0]], rows, sem)

    def g_wait(si, rows, sem):
      pltpu.make_async_copy(y_hbm.at[si.at[0]], rows, sem).wait()

    def do_list(r):
      pltpu.sync_copy(cnts_hbm.at[r], cb)
      cnt = cb[0, pl.ds(0, LANES)][0]
      nb = (cnt + B - 1) // B
      srow = srcl_hbm.at[r]
      drow = dstl_hbm.at[r]

      def load(j, si, di):
        pltpu.sync_copy(srow.at[j], si)
        pltpu.sync_copy(drow.at[j], di)

      @pl.loop(0, MAXNB)
      def _(k):
        @pl.when(k < nb)
        def _():
          load(k, si0, di0)
          pltpu.sync_copy(y_hbm.at[si0.at[0]], rows0)
          pltpu.sync_copy(rows0, acc.at[di0.at[0]], add=True)

    do_list((2 * sid) * 2 + cid)
    do_list((2 * sid + 1) * 2 + cid)

    plsc.subcore_barrier()
    pltpu.sync_copy(acc.at[pl.ds(roff, CH)],
                    out_hbm.at[cid].at[pl.ds(roff, CH)])

  zero = jnp.zeros((NR, D), jnp.float32)
  return seg_kernel(y, srcl, dstl, cnts, zero)


def _sc_segsum_scalar(z, src3, dst3):
  @pl.kernel(
      out_type=jax.ShapeDtypeStruct((NW, 1, N), jnp.float32),
      mesh=_vec_mesh,
      scratch_types=[
          pltpu.VMEM((1, N), jnp.float32),
          pltpu.VMEM((1, EPW), jnp.int32),
          pltpu.VMEM((1, EPW), jnp.int32),
          pltpu.VMEM((1, N), jnp.float32),
      ],
      compiler_params=_sc_params,
  )
  def seg2_kernel(z_hbm, src_hbm, dst_hbm, out_hbm, zv, sv, dv, acc):
    cid = lax.axis_index("c")
    sid = lax.axis_index("s")
    wid = sid * NC + cid

    pltpu.sync_copy(z_hbm, zv)
    pltpu.sync_copy(src_hbm.at[wid], sv)
    pltpu.sync_copy(dst_hbm.at[wid], dv)

    zeros = jnp.zeros((LANES,), jnp.float32)

    @pl.loop(0, N, step=LANES)
    def _(i):
      acc[0, pl.ds(i, LANES)] = zeros

    @pl.loop(0, EPW, step=LANES)
    def _(i):
      s = sv[0, pl.ds(i, LANES)]
      d = dv[0, pl.ds(i, LANES)]
      vals = plsc.load_gather(zv.at[0], [s])
      plsc.addupdate_scatter(acc.at[0], [d], vals)

    pltpu.sync_copy(acc, out_hbm.at[wid])

  return seg2_kernel(z, src3, dst3)


R = 1000


def _tc_norms(doutp, dinp):
  def body(doutp_ref, dinp_ref, no_ref, ni_ref, nir_ref):
    no = lax.rsqrt(jnp.clip(jnp.sum(doutp_ref[...], axis=0), 1.0, None))
    ni = lax.rsqrt(jnp.clip(jnp.sum(dinp_ref[...], axis=0), 1.0, None))
    nir_ref[...] = ni[None, :]
    no_ref[...] = no[:, None]
    ni_ref[...] = ni[:, None]

  return pl.pallas_call(
      body,
      in_specs=[
          pl.BlockSpec((NW, N), lambda: (0, 0)),
          pl.BlockSpec((NW, N), lambda: (0, 0)),
      ],
      out_specs=[
          pl.BlockSpec((N, 1), lambda: (0, 0)),
          pl.BlockSpec((N, 1), lambda: (0, 0)),
          pl.BlockSpec((1, N), lambda: (0, 0)),
      ],
      out_shape=[
          jax.ShapeDtypeStruct((N, 1), jnp.float32),
          jax.ShapeDtypeStruct((N, 1), jnp.float32),
          jax.ShapeDtypeStruct((1, N), jnp.float32),
      ],
  )(doutp, dinp)


def _tc_mm1(x, w1, nout):
  def body(x_ref, w1_ref, no_ref, y_ref):
    y_ref[...] = (
        jnp.dot(x_ref[...], w1_ref[...], preferred_element_type=jnp.float32)
        * no_ref[...])

  return pl.pallas_call(
      body,
      grid=(N // R,),
      in_specs=[
          pl.BlockSpec((R, D), lambda i: (i, 0)),
          pl.BlockSpec((D, D), lambda i: (0, 0)),
          pl.BlockSpec((R, 1), lambda i: (i, 0)),
      ],
      out_specs=pl.BlockSpec((R, D), lambda i: (i, 0)),
      out_shape=jax.ShapeDtypeStruct((N, D), jnp.float32),
  )(x, w1, nout)


R2 = 1000


def _tc_mm2(aggp, nin, nout, b1, w2t):
  bpc = NHALF // R2

  def body(agg_ref, ni_ref, no_ref, b1_ref, w2_ref, z_ref):
    h = jnp.maximum(agg_ref[0] * ni_ref[...] + b1_ref[...], 0.0)
    z_ref[...] = jnp.sum(h * w2_ref[...], axis=1, keepdims=True) * no_ref[...]

  return pl.pallas_call(
      body,
      grid=(N // R2,),
      in_specs=[
          pl.BlockSpec((1, R2, D), lambda i: (i // bpc, i % bpc, 0)),
          pl.BlockSpec((R2, 1), lambda i: (i, 0)),
          pl.BlockSpec((R2, 1), lambda i: (i, 0)),
          pl.BlockSpec((1, D), lambda i: (0, 0)),
          pl.BlockSpec((1, D), lambda i: (0, 0)),
      ],
      out_specs=pl.BlockSpec((R2, 1), lambda i: (i, 0)),
      out_shape=jax.ShapeDtypeStruct((N, 1), jnp.float32),
  )(aggp, nin, nout, b1, w2t)


def _tc_out(a2p, nin_row, b2):
  def body(a2p_ref, ni_ref, b2_ref, o_ref):
    s = jnp.sum(a2p_ref[...], axis=0, keepdims=True)
    o_ref[...] = jax.nn.sigmoid(s * ni_ref[...] + b2_ref[0, 0])

  return pl.pallas_call(
      body,
      in_specs=[
          pl.BlockSpec((NW, N), lambda: (0, 0)),
          pl.BlockSpec((1, N), lambda: (0, 0)),
          pl.BlockSpec((1, 1), lambda: (0, 0)),
      ],
      out_specs=pl.BlockSpec((1, N), lambda: (0, 0)),
      out_shape=jax.ShapeDtypeStruct((1, N), jnp.float32),
  )(a2p, nin_row, b2)


def kernel(x, edge_index, W1, b1, W2, b2):
  src = edge_index[0].astype(jnp.int32)
  dst = edge_index[1].astype(jnp.int32)
  src3 = src.reshape(NW, 1, EPW)
  dst3 = dst.reshape(NW, 1, EPW)

  doutp, dinp, srcl, dstl, cnts = _sc_degrees(src3, dst3)
  nout, nin, nin_row = _tc_norms(doutp.reshape(NW, N), dinp.reshape(NW, N))
  y = _tc_mm1(x, W1, nout)
  aggp = _sc_segsum_rows(y, srcl, dstl, cnts)
  z = _tc_mm2(aggp, nin, nout, b1.reshape(1, D), W2.reshape(1, D))
  a2p = _sc_segsum_scalar(z.reshape(1, N), src3, dst3)
  out = _tc_out(a2p.reshape(NW, N), nin_row, b2.reshape(1, 1))
  return out.reshape(N, 1)


# speedup vs baseline: 1.0157x; 1.0157x over previous
import dataclasses

import jax
import jax.numpy as jnp
from jax import lax
from jax.experimental import pallas as pl
from jax.experimental.pallas import tpu as pltpu
from jax.experimental.pallas import tpu_sc as plsc

N = 10000
E = 320000
D = 128

NC = 2
NS = 16
NW = NC * NS
LANES = 16

EPW = E // NW
B = 512
NHALF = 5000
NR = 5120
DUMP = NHALF
CH = NR // NS
K = 10752
MAXNB = K // B
NL = NW * NC

_vec_mesh = plsc.VectorSubcoreMesh(
    core_axis_name="c", subcore_axis_name="s", num_cores=NC, num_subcores=NS
)

_sc_params = pltpu.CompilerParams()
if "needs_layout_passes" in pltpu.CompilerParams.__dataclass_fields__:
  _sc_params = dataclasses.replace(_sc_params, needs_layout_passes=False)


def _sc_degrees(src3, dst3):
  @pl.kernel(
      out_type=(
          jax.ShapeDtypeStruct((NW, 1, N), jnp.float32),
          jax.ShapeDtypeStruct((NW, 1, N), jnp.float32),
          jax.ShapeDtypeStruct((NL, MAXNB, 1, B), jnp.int32),
          jax.ShapeDtypeStruct((NL, MAXNB, 1, B), jnp.int32),
          jax.ShapeDtypeStruct((NL, 1, 16), jnp.int32),
      ),
      mesh=_vec_mesh,
      scratch_types=[
          pltpu.VMEM((1, EPW), jnp.int32),
          pltpu.VMEM((1, EPW), jnp.int32),
          pltpu.VMEM((1, N), jnp.float32),
          pltpu.VMEM((1, N), jnp.float32),
          pltpu.VMEM((MAXNB, 1, B), jnp.int32),
          pltpu.VMEM((MAXNB, 1, B), jnp.int32),
          pltpu.VMEM((MAXNB, 1, B), jnp.int32),
          pltpu.VMEM((MAXNB, 1, B), jnp.int32),
          pltpu.VMEM((1, 16), jnp.int32),
      ],
      compiler_params=_sc_params,
  )
  def deg_kernel(src_hbm, dst_hbm, dout_hbm, din_hbm, srcl_hbm, dstl_hbm,
                 cnts_hbm, sv, dv, aout, ain, sl0, dl0, sl1, dl1, cb):
    cid = lax.axis_index("c")
    sid = lax.axis_index("s")
    wid = sid * NC + cid

    pltpu.sync_copy(src_hbm.at[wid], sv)
    pltpu.sync_copy(dst_hbm.at[wid], dv)

    zeros = jnp.zeros((LANES,), jnp.float32)
    ones = jnp.ones((LANES,), jnp.float32)
    izeros = jnp.zeros((LANES,), jnp.int32)
    iones = jnp.ones((LANES,), jnp.int32)
    idump = jnp.full((LANES,), DUMP, jnp.int32)

    @pl.loop(0, N, step=LANES)
    def _(i):
      aout[0, pl.ds(i, LANES)] = zeros
      ain[0, pl.ds(i, LANES)] = zeros

    @pl.loop(0, MAXNB)
    def _(j):
      @pl.loop(0, B, step=LANES)
      def _(l):
        sl0[j, 0, pl.ds(l, LANES)] = izeros
        dl0[j, 0, pl.ds(l, LANES)] = idump
        sl1[j, 0, pl.ds(l, LANES)] = izeros
        dl1[j, 0, pl.ds(l, LANES)] = idump

    def edge_body(i, carry):
      off0, off1 = carry
      s = sv[0, pl.ds(i, LANES)]
      d = dv[0, pl.ds(i, LANES)]
      plsc.addupdate_scatter(aout.at[0], [s], ones)
      plsc.addupdate_scatter(ain.at[0], [d], ones)

      izero_v = jnp.zeros((LANES,), jnp.int32)

      m0 = d < NHALF
      c0 = plsc.cumsum(iones, mask=m0)
      pos0 = off0 + c0 - 1
      j0 = jnp.right_shift(pos0, 9)
      l0 = jnp.bitwise_and(pos0, B - 1)
      plsc.store_scatter(sl0, [j0, izero_v, l0], s, mask=m0)
      plsc.store_scatter(dl0, [j0, izero_v, l0], d, mask=m0)
      n0 = plsc.all_reduce_population_count(m0)

      m1 = jnp.logical_not(m0)
      c1 = plsc.cumsum(iones, mask=m1)
      pos1 = off1 + c1 - 1
      j1 = jnp.right_shift(pos1, 9)
      l1 = jnp.bitwise_and(pos1, B - 1)
      plsc.store_scatter(sl1, [j1, izero_v, l1], s, mask=m1)
      plsc.store_scatter(dl1, [j1, izero_v, l1], d - NHALF, mask=m1)
      n1 = plsc.all_reduce_population_count(m1)
      return (off0 + n0.astype(jnp.int32), off1 + n1.astype(jnp.int32))

    off0, off1 = pl.loop(0, EPW, step=LANES,
                         init_carry=(izeros, izeros))(edge_body)
    pltpu.sync_copy(aout, dout_hbm.at[wid])
    pltpu.sync_copy(ain, din_hbm.at[wid])
    pltpu.sync_copy(sl0, srcl_hbm.at[2 * wid])
    pltpu.sync_copy(dl0, dstl_hbm.at[2 * wid])
    pltpu.sync_copy(sl1, srcl_hbm.at[2 * wid + 1])
    pltpu.sync_copy(dl1, dstl_hbm.at[2 * wid + 1])
    cb[0, pl.ds(0, LANES)] = off0
    pltpu.sync_copy(cb, cnts_hbm.at[2 * wid])
    cb[0, pl.ds(0, LANES)] = off1
    pltpu.sync_copy(cb, cnts_hbm.at[2 * wid + 1])

  return deg_kernel(src3, dst3)


def _sc_segsum_rows(y, srcl, dstl, cnts):
  @pl.kernel(
      out_type=jax.ShapeDtypeStruct((NC, NR, D), jnp.float32),
      mesh=_vec_mesh,
      scratch_types=[
          pltpu.VMEM((1, B), jnp.int32),
          pltpu.VMEM((1, B), jnp.int32),
          pltpu.VMEM((B, D), jnp.float32),
          pltpu.VMEM_SHARED((NR, D), jnp.float32),
          pltpu.VMEM((1, 16), jnp.int32),
          pltpu.SemaphoreType.DMA,
      ],
      compiler_params=_sc_params,
  )
  def seg_kernel(y_hbm, srcl_hbm, dstl_hbm, cnts_hbm, zero_hbm, out_hbm,
                 si0, di0, rows0, acc, cb, sem0):
    cid = lax.axis_index("c")
    sid = lax.axis_index("s")

    roff = pl.multiple_of(sid * CH, 8)
    pltpu.sync_copy(zero_hbm.at[pl.ds(roff, CH)], acc.at[pl.ds(roff, CH)])
    plsc.subcore_barrier()

    def g_start(si, rows, sem):
      pltpu.async_copy(y_hbm.at[si.at[0]], rows, sem)

    def g_wait(si, rows, sem):
      pltpu.make_async_copy(y_hbm.at[si.at[0]], rows, sem).wait()

    def do_list(r):
      pltpu.sync_copy(cnts_hbm.at[r], cb)
      cnt = cb[0, pl.ds(0, LANES)][0]
      nb = (cnt + B - 1) // B
      srow = srcl_hbm.at[r]
      drow = dstl_hbm.at[r]

      def load(j, si, di):
        pltpu.sync_copy(srow.at[j], si)
        pltpu.sync_copy(drow.at[j], di)

      @pl.loop(0, MAXNB)
      def _(k):
        @pl.when(k < nb)
        def _():
          load(k, si0, di0)
          pltpu.sync_copy(y_hbm.at[si0.at[0]], rows0)
          pltpu.sync_copy(rows0, acc.at[di0.at[0]], add=True)

    do_list((2 * sid) * 2 + cid)
    do_list((2 * sid + 1) * 2 + cid)

    plsc.subcore_barrier()
    pltpu.sync_copy(acc.at[pl.ds(roff, CH)],
                    out_hbm.at[cid].at[pl.ds(roff, CH)])

  zero = jnp.zeros((NR, D), jnp.float32)
  return seg_kernel(y, srcl, dstl, cnts, zero)


def _sc_segsum_scalar(z, src3, dst3):
  @pl.kernel(
      out_type=jax.ShapeDtypeStruct((NW, 1, N), jnp.float32),
      mesh=_vec_mesh,
      scratch_types=[
          pltpu.VMEM((1, N), jnp.float32),
          pltpu.VMEM((1, EPW), jnp.int32),
          pltpu.VMEM((1, EPW), jnp.int32),
          pltpu.VMEM((1, N), jnp.float32),
      ],
      compiler_params=_sc_params,
  )
  def seg2_kernel(z_hbm, src_hbm, dst_hbm, out_hbm, zv, sv, dv, acc):
    cid = lax.axis_index("c")
    sid = lax.axis_index("s")
    wid = sid * NC + cid

    pltpu.sync_copy(z_hbm, zv)
    pltpu.sync_copy(src_hbm.at[wid], sv)
    pltpu.sync_copy(dst_hbm.at[wid], dv)

    zeros = jnp.zeros((LANES,), jnp.float32)

    @pl.loop(0, N, step=LANES)
    def _(i):
      acc[0, pl.ds(i, LANES)] = zeros

    @pl.loop(0, EPW, step=LANES)
    def _(i):
      s = sv[0, pl.ds(i, LANES)]
      d = dv[0, pl.ds(i, LANES)]
      vals = plsc.load_gather(zv.at[0], [s])
      plsc.addupdate_scatter(acc.at[0], [d], vals)

    pltpu.sync_copy(acc, out_hbm.at[wid])

  return seg2_kernel(z, src3, dst3)


R = 1000


def _tc_norms(doutp, dinp):
  def body(doutp_ref, dinp_ref, no_ref, ni_ref, nir_ref):
    no = lax.rsqrt(jnp.clip(jnp.sum(doutp_ref[...], axis=0), 1.0, None))
    ni = lax.rsqrt(jnp.clip(jnp.sum(dinp_ref[...], axis=0), 1.0, None))
    nir_ref[...] = ni[None, :]
    no_ref[...] = no[:, None]
    ni_ref[...] = ni[:, None]

  return pl.pallas_call(
      body,
      in_specs=[
          pl.BlockSpec((NW, N), lambda: (0, 0)),
          pl.BlockSpec((NW, N), lambda: (0, 0)),
      ],
      out_specs=[
          pl.BlockSpec((N, 1), lambda: (0, 0)),
          pl.BlockSpec((N, 1), lambda: (0, 0)),
          pl.BlockSpec((1, N), lambda: (0, 0)),
      ],
      out_shape=[
          jax.ShapeDtypeStruct((N, 1), jnp.float32),
          jax.ShapeDtypeStruct((N, 1), jnp.float32),
          jax.ShapeDtypeStruct((1, N), jnp.float32),
      ],
  )(doutp, dinp)


def _tc_mm1(x, w1, nout):
  def body(x_ref, w1_ref, no_ref, y_ref):
    y_ref[...] = (
        jnp.dot(x_ref[...], w1_ref[...], preferred_element_type=jnp.float32)
        * no_ref[...])

  return pl.pallas_call(
      body,
      grid=(N // R,),
      in_specs=[
          pl.BlockSpec((R, D), lambda i: (i, 0)),
          pl.BlockSpec((D, D), lambda i: (0, 0)),
          pl.BlockSpec((R, 1), lambda i: (i, 0)),
      ],
      out_specs=pl.BlockSpec((R, D), lambda i: (i, 0)),
      out_shape=jax.ShapeDtypeStruct((N, D), jnp.float32),
  )(x, w1, nout)


R2 = 1000


def _tc_mm2(aggp, nin, nout, b1, w2t):
  bpc = NHALF // R2

  def body(agg_ref, ni_ref, no_ref, b1_ref, w2_ref, z_ref):
    h = jnp.maximum(agg_ref[0] * ni_ref[...] + b1_ref[...], 0.0)
    z_ref[...] = jnp.sum(h * w2_ref[...], axis=1, keepdims=True) * no_ref[...]

  return pl.pallas_call(
      body,
      grid=(N // R2,),
      in_specs=[
          pl.BlockSpec((1, R2, D), lambda i: (i // bpc, i % bpc, 0)),
          pl.BlockSpec((R2, 1), lambda i: (i, 0)),
          pl.BlockSpec((R2, 1), lambda i: (i, 0)),
          pl.BlockSpec((1, D), lambda i: (0, 0)),
          pl.BlockSpec((1, D), lambda i: (0, 0)),
      ],
      out_specs=pl.BlockSpec((R2, 1), lambda i: (i, 0)),
      out_shape=jax.ShapeDtypeStruct((N, 1), jnp.float32),
  )(aggp, nin, nout, b1, w2t)


def _tc_out(a2p, nin_row, b2):
  def body(a2p_ref, ni_ref, b2_ref, o_ref):
    s = jnp.sum(a2p_ref[...], axis=0, keepdims=True)
    o_ref[...] = jax.nn.sigmoid(s * ni_ref[...] + b2_ref[0, 0])

  return pl.pallas_call(
      body,
      in_specs=[
          pl.BlockSpec((NW, N), lambda: (0, 0)),
          pl.BlockSpec((1, N), lambda: (0, 0)),
          pl.BlockSpec((1, 1), lambda: (0, 0)),
      ],
      out_specs=pl.BlockSpec((1, N), lambda: (0, 0)),
      out_shape=jax.ShapeDtypeStruct((1, N), jnp.float32),
  )(a2p, nin_row, b2)


def kernel(x, edge_index, W1, b1, W2, b2):
  src = edge_index[0].astype(jnp.int32)
  dst = edge_index[1].astype(jnp.int32)
  src3 = src.reshape(NW, 1, EPW)
  dst3 = dst.reshape(NW, 1, EPW)

  doutp, dinp, srcl, dstl, cnts = _sc_degrees(src3, dst3)
  nout, nin, nin_row = _tc_norms(doutp.reshape(NW, N), dinp.reshape(NW, N))
  y = _tc_mm1(x, W1, nout)
  aggp = _sc_segsum_rows(y, srcl, dstl, cnts)
  z = _tc_mm2(aggp, nin, nout, b1.reshape(1, D), W2.reshape(1, D))
  a2p = _sc_segsum_scalar(z.reshape(1, N), src3, dst3)
  out = _tc_out(a2p.reshape(NW, N), nin_row, b2.reshape(1, 1))
  return out.reshape(N, 1)


# speedup vs baseline: 1.5166x; 1.4931x over previous
import dataclasses

import jax
import jax.numpy as jnp
from jax import lax
from jax.experimental import pallas as pl
from jax.experimental.pallas import tpu as pltpu
from jax.experimental.pallas import tpu_sc as plsc

N = 10000
E = 320000
D = 128

NC = 2
NS = 16
NW = NC * NS
LANES = 16

EPW = E // NW
B = 200
NB = E // B
GMAX = NB // NS
NHALF = 5000
NR = 5120
DUMP = NHALF
CH = NR // NS

_vec_mesh = plsc.VectorSubcoreMesh(
    core_axis_name="c", subcore_axis_name="s", num_cores=NC, num_subcores=NS
)

_sc_params = pltpu.CompilerParams()
if "needs_layout_passes" in pltpu.CompilerParams.__dataclass_fields__:
  _sc_params = dataclasses.replace(_sc_params, needs_layout_passes=False)


def _sc_degrees(src3, dst3):
  @pl.kernel(
      out_type=(
          jax.ShapeDtypeStruct((NW, 1, N), jnp.float32),
          jax.ShapeDtypeStruct((NW, 1, N), jnp.float32),
      ),
      mesh=_vec_mesh,
      scratch_types=[
          pltpu.VMEM((1, EPW), jnp.int32),
          pltpu.VMEM((1, EPW), jnp.int32),
          pltpu.VMEM((1, N), jnp.float32),
          pltpu.VMEM((1, N), jnp.float32),
      ],
      compiler_params=_sc_params,
  )
  def deg_kernel(src_hbm, dst_hbm, dout_hbm, din_hbm, sv, dv, aout, ain):
    cid = lax.axis_index("c")
    sid = lax.axis_index("s")
    wid = sid * NC + cid

    pltpu.sync_copy(src_hbm.at[wid], sv)
    pltpu.sync_copy(dst_hbm.at[wid], dv)

    zeros = jnp.zeros((LANES,), jnp.float32)
    ones = jnp.ones((LANES,), jnp.float32)

    @pl.loop(0, N, step=LANES)
    def _(i):
      aout[0, pl.ds(i, LANES)] = zeros
      ain[0, pl.ds(i, LANES)] = zeros

    @pl.loop(0, EPW, step=LANES)
    def _(i):
      s = sv[0, pl.ds(i, LANES)]
      d = dv[0, pl.ds(i, LANES)]
      plsc.addupdate_scatter(aout.at[0], [s], ones)
      plsc.addupdate_scatter(ain.at[0], [d], ones)

    pltpu.sync_copy(aout, dout_hbm.at[wid])
    pltpu.sync_copy(ain, din_hbm.at[wid])

  return deg_kernel(src3, dst3)


def _sc_segsum_rows(y, srcb3, dstb3):
  @pl.kernel(
      out_type=jax.ShapeDtypeStruct((NC, NR, D), jnp.float32),
      mesh=_vec_mesh,
      scratch_types=[
          pltpu.VMEM((1, B), jnp.int32),
          pltpu.VMEM((1, B), jnp.int32),
          pltpu.VMEM((1, B), jnp.int32),
          pltpu.VMEM((1, B), jnp.int32),
          pltpu.VMEM((1, B), jnp.int32),
          pltpu.VMEM((1, B), jnp.int32),
          pltpu.VMEM((B, D), jnp.float32),
          pltpu.VMEM((B, D), jnp.float32),
          pltpu.VMEM_SHARED((NR, D), jnp.float32),
          pltpu.SemaphoreType.DMA,
          pltpu.SemaphoreType.DMA,
      ],
      compiler_params=_sc_params,
  )
  def seg_kernel(y_hbm, srcb_hbm, dstb_hbm, zero_hbm, out_hbm,
                 si0, di0, dr0, si1, di1, dr1, rows0, rows1, acc,
                 sem0, sem1):
    cid = lax.axis_index("c")
    sid = lax.axis_index("s")
    base = cid * NHALF

    roff = pl.multiple_of(sid * CH, 8)
    pltpu.sync_copy(zero_hbm.at[pl.ds(roff, CH)], acc.at[pl.ds(roff, CH)])
    plsc.subcore_barrier()

    def load_idx(b, si, di, dr):
      pltpu.sync_copy(srcb_hbm.at[b], si)
      pltpu.sync_copy(dstb_hbm.at[b], di)

      @pl.loop(0, B, step=LANES)
      def _(j):
        d = di[0, pl.ds(j, LANES)] - base
        ok = (d >= 0) & (d < NHALF)
        dr[0, pl.ds(j, LANES)] = jnp.where(ok, d, DUMP)

    def g_start(si, rows, sem):
      pltpu.async_copy(y_hbm.at[si.at[0]], rows, sem)

    def g_wait(si, rows, sem):
      pltpu.make_async_copy(y_hbm.at[si.at[0]], rows, sem).wait()

    load_idx(sid, si0, di0, dr0)
    g_start(si0, rows0, sem0)

    @pl.loop(0, GMAX, step=2)
    def _(k):
      b1 = (k + 1) * NS + sid
      load_idx(b1, si1, di1, dr1)
      g_start(si1, rows1, sem1)
      g_wait(si0, rows0, sem0)
      pltpu.sync_copy(rows0, acc.at[dr0.at[0]], add=True)

      @pl.when(k + 2 < GMAX)
      def _():
        b2 = (k + 2) * NS + sid
        load_idx(b2, si0, di0, dr0)
        g_start(si0, rows0, sem0)

      g_wait(si1, rows1, sem1)
      pltpu.sync_copy(rows1, acc.at[dr1.at[0]], add=True)

    plsc.subcore_barrier()
    pltpu.sync_copy(acc.at[pl.ds(roff, CH)],
                    out_hbm.at[cid].at[pl.ds(roff, CH)])

  zero = jnp.zeros((NR, D), jnp.float32)
  return seg_kernel(y, srcb3, dstb3, zero)


def _sc_segsum_scalar(z, src3, dst3):
  @pl.kernel(
      out_type=jax.ShapeDtypeStruct((NW, 1, N), jnp.float32),
      mesh=_vec_mesh,
      scratch_types=[
          pltpu.VMEM((1, N), jnp.float32),
          pltpu.VMEM((1, EPW), jnp.int32),
          pltpu.VMEM((1, EPW), jnp.int32),
          pltpu.VMEM((1, N), jnp.float32),
      ],
      compiler_params=_sc_params,
  )
  def seg2_kernel(z_hbm, src_hbm, dst_hbm, out_hbm, zv, sv, dv, acc):
    cid = lax.axis_index("c")
    sid = lax.axis_index("s")
    wid = sid * NC + cid

    pltpu.sync_copy(z_hbm, zv)
    pltpu.sync_copy(src_hbm.at[wid], sv)
    pltpu.sync_copy(dst_hbm.at[wid], dv)

    zeros = jnp.zeros((LANES,), jnp.float32)

    @pl.loop(0, N, step=LANES)
    def _(i):
      acc[0, pl.ds(i, LANES)] = zeros

    @pl.loop(0, EPW, step=LANES)
    def _(i):
      s = sv[0, pl.ds(i, LANES)]
      d = dv[0, pl.ds(i, LANES)]
      vals = plsc.load_gather(zv.at[0], [s])
      plsc.addupdate_scatter(acc.at[0], [d], vals)

    pltpu.sync_copy(acc, out_hbm.at[wid])

  return seg2_kernel(z, src3, dst3)


R = 1000


def _tc_norms(doutp, dinp):
  def body(doutp_ref, dinp_ref, no_ref, ni_ref, nir_ref):
    no = lax.rsqrt(jnp.clip(jnp.sum(doutp_ref[...], axis=0), 1.0, None))
    ni = lax.rsqrt(jnp.clip(jnp.sum(dinp_ref[...], axis=0), 1.0, None))
    nir_ref[...] = ni[None, :]
    no_ref[...] = no[:, None]
    ni_ref[...] = ni[:, None]

  return pl.pallas_call(
      body,
      in_specs=[
          pl.BlockSpec((NW, N), lambda: (0, 0)),
          pl.BlockSpec((NW, N), lambda: (0, 0)),
      ],
      out_specs=[
          pl.BlockSpec((N, 1), lambda: (0, 0)),
          pl.BlockSpec((N, 1), lambda: (0, 0)),
          pl.BlockSpec((1, N), lambda: (0, 0)),
      ],
      out_shape=[
          jax.ShapeDtypeStruct((N, 1), jnp.float32),
          jax.ShapeDtypeStruct((N, 1), jnp.float32),
          jax.ShapeDtypeStruct((1, N), jnp.float32),
      ],
  )(doutp, dinp)


def _tc_mm1(x, w1, nout):
  def body(x_ref, w1_ref, no_ref, y_ref):
    y_ref[...] = (
        jnp.dot(x_ref[...], w1_ref[...], preferred_element_type=jnp.float32)
        * no_ref[...])

  return pl.pallas_call(
      body,
      grid=(N // R,),
      in_specs=[
          pl.BlockSpec((R, D), lambda i: (i, 0)),
          pl.BlockSpec((D, D), lambda i: (0, 0)),
          pl.BlockSpec((R, 1), lambda i: (i, 0)),
      ],
      out_specs=pl.BlockSpec((R, D), lambda i: (i, 0)),
      out_shape=jax.ShapeDtypeStruct((N, D), jnp.float32),
  )(x, w1, nout)


R2 = 1000


def _tc_mm2(aggp, nin, nout, b1, w2t):
  bpc = NHALF // R2

  def body(agg_ref, ni_ref, no_ref, b1_ref, w2_ref, z_ref):
    h = jnp.maximum(agg_ref[0] * ni_ref[...] + b1_ref[...], 0.0)
    z_ref[...] = jnp.sum(h * w2_ref[...], axis=1, keepdims=True) * no_ref[...]

  return pl.pallas_call(
      body,
      grid=(N // R2,),
      in_specs=[
          pl.BlockSpec((1, R2, D), lambda i: (i // bpc, i % bpc, 0)),
          pl.BlockSpec((R2, 1), lambda i: (i, 0)),
          pl.BlockSpec((R2, 1), lambda i: (i, 0)),
          pl.BlockSpec((1, D), lambda i: (0, 0)),
          pl.BlockSpec((1, D), lambda i: (0, 0)),
      ],
      out_specs=pl.BlockSpec((R2, 1), lambda i: (i, 0)),
      out_shape=jax.ShapeDtypeStruct((N, 1), jnp.float32),
  )(aggp, nin, nout, b1, w2t)


def _tc_out(a2p, nin_row, b2):
  def body(a2p_ref, ni_ref, b2_ref, o_ref):
    s = jnp.sum(a2p_ref[...], axis=0, keepdims=True)
    o_ref[...] = jax.nn.sigmoid(s * ni_ref[...] + b2_ref[0, 0])

  return pl.pallas_call(
      body,
      in_specs=[
          pl.BlockSpec((NW, N), lambda: (0, 0)),
          pl.BlockSpec((1, N), lambda: (0, 0)),
          pl.BlockSpec((1, 1), lambda: (0, 0)),
      ],
      out_specs=pl.BlockSpec((1, N), lambda: (0, 0)),
      out_shape=jax.ShapeDtypeStruct((1, N), jnp.float32),
  )(a2p, nin_row, b2)


def kernel(x, edge_index, W1, b1, W2, b2):
  src = edge_index[0].astype(jnp.int32)
  dst = edge_index[1].astype(jnp.int32)
  srcb3 = src.reshape(NB, 1, B)
  dstb3 = dst.reshape(NB, 1, B)
  src3 = src.reshape(NW, 1, EPW)
  dst3 = dst.reshape(NW, 1, EPW)

  doutp, dinp = _sc_degrees(src3, dst3)
  nout, nin, nin_row = _tc_norms(doutp.reshape(NW, N), dinp.reshape(NW, N))
  y = _tc_mm1(x, W1, nout)
  aggp = _sc_segsum_rows(y, srcb3, dstb3)
  z = _tc_mm2(aggp, nin, nout, b1.reshape(1, D), W2.reshape(1, D))
  a2p = _sc_segsum_scalar(z.reshape(1, N), src3, dst3)
  out = _tc_out(a2p.reshape(NW, N), nin_row, b2.reshape(1, 1))
  return out.reshape(N, 1)


# speedup vs baseline: 1.5203x; 1.0024x over previous
import dataclasses

import jax
import jax.numpy as jnp
from jax import lax
from jax.experimental import pallas as pl
from jax.experimental.pallas import tpu as pltpu
from jax.experimental.pallas import tpu_sc as plsc

N = 10000
E = 320000
D = 128

NC = 2
NS = 16
NW = NC * NS
LANES = 16

EPW = E // NW
B = 200
NB = E // B
GMAX = NB // NS
NHALF = 5000
NR = 5120
DUMP = NHALF
CH = NR // NS

_vec_mesh = plsc.VectorSubcoreMesh(
    core_axis_name="c", subcore_axis_name="s", num_cores=NC, num_subcores=NS
)

_sc_params = pltpu.CompilerParams()
if "needs_layout_passes" in pltpu.CompilerParams.__dataclass_fields__:
  _sc_params = dataclasses.replace(_sc_params, needs_layout_passes=False)


def _sc_degrees(src3, dst3):
  @pl.kernel(
      out_type=(
          jax.ShapeDtypeStruct((NW, 1, N), jnp.float32),
          jax.ShapeDtypeStruct((NW, 1, N), jnp.float32),
      ),
      mesh=_vec_mesh,
      scratch_types=[
          pltpu.VMEM((1, EPW), jnp.int32),
          pltpu.VMEM((1, EPW), jnp.int32),
          pltpu.VMEM((1, N), jnp.float32),
          pltpu.VMEM((1, N), jnp.float32),
      ],
      compiler_params=_sc_params,
  )
  def deg_kernel(src_hbm, dst_hbm, dout_hbm, din_hbm, sv, dv, aout, ain):
    cid = lax.axis_index("c")
    sid = lax.axis_index("s")
    wid = sid * NC + cid

    pltpu.sync_copy(src_hbm.at[wid], sv)
    pltpu.sync_copy(dst_hbm.at[wid], dv)

    zeros = jnp.zeros((LANES,), jnp.float32)
    ones = jnp.ones((LANES,), jnp.float32)

    @pl.loop(0, N, step=LANES)
    def _(i):
      aout[0, pl.ds(i, LANES)] = zeros
      ain[0, pl.ds(i, LANES)] = zeros

    @pl.loop(0, EPW, step=LANES)
    def _(i):
      s = sv[0, pl.ds(i, LANES)]
      d = dv[0, pl.ds(i, LANES)]
      plsc.addupdate_scatter(aout.at[0], [s], ones)
      plsc.addupdate_scatter(ain.at[0], [d], ones)

    pltpu.sync_copy(aout, dout_hbm.at[wid])
    pltpu.sync_copy(ain, din_hbm.at[wid])

  return deg_kernel(src3, dst3)


def _sc_segsum_rows(y, srcb3, dstb3):
  @pl.kernel(
      out_type=jax.ShapeDtypeStruct((NC, NR, D), jnp.float32),
      mesh=_vec_mesh,
      scratch_types=[
          pltpu.VMEM((3, 1, B), jnp.int32),
          pltpu.VMEM((3, 1, B), jnp.int32),
          pltpu.VMEM((3, 1, B), jnp.int32),
          pltpu.VMEM((B, D), jnp.float32),
          pltpu.VMEM((B, D), jnp.float32),
          pltpu.VMEM((B, D), jnp.float32),
          pltpu.VMEM_SHARED((NR, D), jnp.float32),
          pltpu.SemaphoreType.DMA,
          pltpu.SemaphoreType.DMA,
          pltpu.SemaphoreType.DMA,
          pltpu.SemaphoreType.DMA,
          pltpu.SemaphoreType.DMA,
          pltpu.SemaphoreType.DMA,
      ],
      compiler_params=_sc_params,
  )
  def seg_kernel(y_hbm, srcb_hbm, dstb_hbm, zero_hbm, out_hbm,
                 si, di, dr, rows0, rows1, rows2, acc,
                 gs0, gs1, gs2, ss0, ss1, ss2):
    cid = lax.axis_index("c")
    sid = lax.axis_index("s")
    base = cid * NHALF
    rows = (rows0, rows1, rows2)
    gsem = (gs0, gs1, gs2)
    ssem = (ss0, ss1, ss2)

    roff = pl.multiple_of(sid * CH, 8)
    pltpu.sync_copy(zero_hbm.at[pl.ds(roff, CH)], acc.at[pl.ds(roff, CH)])
    plsc.subcore_barrier()

    def load_idx(k, s):
      b = k * NS + sid
      pltpu.sync_copy(srcb_hbm.at[b], si.at[s])
      pltpu.sync_copy(dstb_hbm.at[b], di.at[s])

      @pl.loop(0, B, step=LANES)
      def _(j):
        d = di[s, 0, pl.ds(j, LANES)] - base
        ok = (d >= 0) & (d < NHALF)
        dr[s, 0, pl.ds(j, LANES)] = jnp.where(ok, d, DUMP)

    def g_start(s):
      pltpu.async_copy(y_hbm.at[si.at[s].at[0]], rows[s], gsem[s])

    def g_wait(s):
      pltpu.make_async_copy(y_hbm.at[si.at[s].at[0]], rows[s],
                            gsem[s]).wait()

    def s_start(s):
      pltpu.async_copy(rows[s], acc.at[dr.at[s].at[0]], ssem[s], add=True)

    def s_wait(s):
      pltpu.make_async_copy(rows[s], acc.at[dr.at[s].at[0]],
                            ssem[s]).wait()

    load_idx(0, 0)
    g_start(0)
    load_idx(1, 1)
    g_start(1)

    @pl.loop(0, GMAX + 2, step=3)
    def _(k):
      for j in range(3):
        b = k + j
        s = j
        s2 = (j + 2) % 3

        @pl.when(b < GMAX)
        def _():
          @pl.when(b + 2 < GMAX)
          def _():
            @pl.when(b >= 1)
            def _():
              s_wait(s2)

            load_idx(b + 2, s2)
            g_start(s2)

          g_wait(s)
          s_start(s)

    s_wait((GMAX - 1) % 3)
    s_wait((GMAX - 2) % 3)
    s_wait((GMAX - 3) % 3)

    plsc.subcore_barrier()
    pltpu.sync_copy(acc.at[pl.ds(roff, CH)],
                    out_hbm.at[cid].at[pl.ds(roff, CH)])

  zero = jnp.zeros((NR, D), jnp.float32)
  return seg_kernel(y, srcb3, dstb3, zero)


def _sc_segsum_scalar(z, src3, dst3):
  @pl.kernel(
      out_type=jax.ShapeDtypeStruct((NW, 1, N), jnp.float32),
      mesh=_vec_mesh,
      scratch_types=[
          pltpu.VMEM((1, N), jnp.float32),
          pltpu.VMEM((1, EPW), jnp.int32),
          pltpu.VMEM((1, EPW), jnp.int32),
          pltpu.VMEM((1, N), jnp.float32),
      ],
      compiler_params=_sc_params,
  )
  def seg2_kernel(z_hbm, src_hbm, dst_hbm, out_hbm, zv, sv, dv, acc):
    cid = lax.axis_index("c")
    sid = lax.axis_index("s")
    wid = sid * NC + cid

    pltpu.sync_copy(z_hbm, zv)
    pltpu.sync_copy(src_hbm.at[wid], sv)
    pltpu.sync_copy(dst_hbm.at[wid], dv)

    zeros = jnp.zeros((LANES,), jnp.float32)

    @pl.loop(0, N, step=LANES)
    def _(i):
      acc[0, pl.ds(i, LANES)] = zeros

    @pl.loop(0, EPW, step=LANES)
    def _(i):
      s = sv[0, pl.ds(i, LANES)]
      d = dv[0, pl.ds(i, LANES)]
      vals = plsc.load_gather(zv.at[0], [s])
      plsc.addupdate_scatter(acc.at[0], [d], vals)

    pltpu.sync_copy(acc, out_hbm.at[wid])

  return seg2_kernel(z, src3, dst3)


R = 1000


def _tc_norms(doutp, dinp):
  def body(doutp_ref, dinp_ref, no_ref, ni_ref, nir_ref):
    no = lax.rsqrt(jnp.clip(jnp.sum(doutp_ref[...], axis=0), 1.0, None))
    ni = lax.rsqrt(jnp.clip(jnp.sum(dinp_ref[...], axis=0), 1.0, None))
    nir_ref[...] = ni[None, :]
    no_ref[...] = no[:, None]
    ni_ref[...] = ni[:, None]

  return pl.pallas_call(
      body,
      in_specs=[
          pl.BlockSpec((NW, N), lambda: (0, 0)),
          pl.BlockSpec((NW, N), lambda: (0, 0)),
      ],
      out_specs=[
          pl.BlockSpec((N, 1), lambda: (0, 0)),
          pl.BlockSpec((N, 1), lambda: (0, 0)),
          pl.BlockSpec((1, N), lambda: (0, 0)),
      ],
      out_shape=[
          jax.ShapeDtypeStruct((N, 1), jnp.float32),
          jax.ShapeDtypeStruct((N, 1), jnp.float32),
          jax.ShapeDtypeStruct((1, N), jnp.float32),
      ],
  )(doutp, dinp)


def _tc_mm1(x, w1, nout):
  def body(x_ref, w1_ref, no_ref, y_ref):
    y_ref[...] = (
        jnp.dot(x_ref[...], w1_ref[...], preferred_element_type=jnp.float32)
        * no_ref[...])

  return pl.pallas_call(
      body,
      grid=(N // R,),
      in_specs=[
          pl.BlockSpec((R, D), lambda i: (i, 0)),
          pl.BlockSpec((D, D), lambda i: (0, 0)),
          pl.BlockSpec((R, 1), lambda i: (i, 0)),
      ],
      out_specs=pl.BlockSpec((R, D), lambda i: (i, 0)),
      out_shape=jax.ShapeDtypeStruct((N, D), jnp.float32),
  )(x, w1, nout)


R2 = 1000


def _tc_mm2(aggp, nin, nout, b1, w2t):
  bpc = NHALF // R2

  def body(agg_ref, ni_ref, no_ref, b1_ref, w2_ref, z_ref):
    h = jnp.maximum(agg_ref[0] * ni_ref[...] + b1_ref[...], 0.0)
    z_ref[...] = jnp.sum(h * w2_ref[...], axis=1, keepdims=True) * no_ref[...]

  return pl.pallas_call(
      body,
      grid=(N // R2,),
      in_specs=[
          pl.BlockSpec((1, R2, D), lambda i: (i // bpc, i % bpc, 0)),
          pl.BlockSpec((R2, 1), lambda i: (i, 0)),
          pl.BlockSpec((R2, 1), lambda i: (i, 0)),
          pl.BlockSpec((1, D), lambda i: (0, 0)),
          pl.BlockSpec((1, D), lambda i: (0, 0)),
      ],
      out_specs=pl.BlockSpec((R2, 1), lambda i: (i, 0)),
      out_shape=jax.ShapeDtypeStruct((N, 1), jnp.float32),
  )(aggp, nin, nout, b1, w2t)


def _tc_out(a2p, nin_row, b2):
  def body(a2p_ref, ni_ref, b2_ref, o_ref):
    s = jnp.sum(a2p_ref[...], axis=0, keepdims=True)
    o_ref[...] = jax.nn.sigmoid(s * ni_ref[...] + b2_ref[0, 0])

  return pl.pallas_call(
      body,
      in_specs=[
          pl.BlockSpec((NW, N), lambda: (0, 0)),
          pl.BlockSpec((1, N), lambda: (0, 0)),
          pl.BlockSpec((1, 1), lambda: (0, 0)),
      ],
      out_specs=pl.BlockSpec((1, N), lambda: (0, 0)),
      out_shape=jax.ShapeDtypeStruct((1, N), jnp.float32),
  )(a2p, nin_row, b2)


def kernel(x, edge_index, W1, b1, W2, b2):
  src = edge_index[0].astype(jnp.int32)
  dst = edge_index[1].astype(jnp.int32)
  srcb3 = src.reshape(NB, 1, B)
  dstb3 = dst.reshape(NB, 1, B)
  src3 = src.reshape(NW, 1, EPW)
  dst3 = dst.reshape(NW, 1, EPW)

  doutp, dinp = _sc_degrees(src3, dst3)
  nout, nin, nin_row = _tc_norms(doutp.reshape(NW, N), dinp.reshape(NW, N))
  y = _tc_mm1(x, W1, nout)
  aggp = _sc_segsum_rows(y, srcb3, dstb3)
  z = _tc_mm2(aggp, nin, nout, b1.reshape(1, D), W2.reshape(1, D))
  a2p = _sc_segsum_scalar(z.reshape(1, N), src3, dst3)
  out = _tc_out(a2p.reshape(NW, N), nin_row, b2.reshape(1, 1))
  return out.reshape(N, 1)


# speedup vs baseline: 1.5621x; 1.0275x over previous
import dataclasses

import jax
import jax.numpy as jnp
from jax import lax
from jax.experimental import pallas as pl
from jax.experimental.pallas import tpu as pltpu
from jax.experimental.pallas import tpu_sc as plsc

N = 10000
E = 320000
D = 128

NC = 2
NS = 16
NW = NC * NS
LANES = 16

EPW = E // NW
B = 200
NB = E // B
GMAX = NB // NS
NHALF = 5000
NR = 5120
DUMP = NHALF
CH = NR // NS

_vec_mesh = plsc.VectorSubcoreMesh(
    core_axis_name="c", subcore_axis_name="s", num_cores=NC, num_subcores=NS
)

_sc_params = pltpu.CompilerParams()
if "needs_layout_passes" in pltpu.CompilerParams.__dataclass_fields__:
  _sc_params = dataclasses.replace(_sc_params, needs_layout_passes=False)


def _sc_degrees(src3, dst3):
  @pl.kernel(
      out_type=(
          jax.ShapeDtypeStruct((NW, 1, N), jnp.float32),
          jax.ShapeDtypeStruct((NW, 1, N), jnp.float32),
      ),
      mesh=_vec_mesh,
      scratch_types=[
          pltpu.VMEM((1, EPW), jnp.int32),
          pltpu.VMEM((1, EPW), jnp.int32),
          pltpu.VMEM((1, N), jnp.float32),
          pltpu.VMEM((1, N), jnp.float32),
      ],
      compiler_params=_sc_params,
  )
  def deg_kernel(src_hbm, dst_hbm, dout_hbm, din_hbm, sv, dv, aout, ain):
    cid = lax.axis_index("c")
    sid = lax.axis_index("s")
    wid = sid * NC + cid

    pltpu.sync_copy(src_hbm.at[wid], sv)
    pltpu.sync_copy(dst_hbm.at[wid], dv)

    zeros = jnp.zeros((LANES,), jnp.float32)
    ones = jnp.ones((LANES,), jnp.float32)

    @pl.loop(0, N, step=LANES)
    def _(i):
      aout[0, pl.ds(i, LANES)] = zeros
      ain[0, pl.ds(i, LANES)] = zeros

    @pl.loop(0, EPW, step=LANES)
    def _(i):
      s = sv[0, pl.ds(i, LANES)]
      d = dv[0, pl.ds(i, LANES)]
      plsc.addupdate_scatter(aout.at[0], [s], ones)
      plsc.addupdate_scatter(ain.at[0], [d], ones)

    pltpu.sync_copy(aout, dout_hbm.at[wid])
    pltpu.sync_copy(ain, din_hbm.at[wid])

  return deg_kernel(src3, dst3)


def _sc_segsum_rows(y, srcb3, dstb3):
  @pl.kernel(
      out_type=jax.ShapeDtypeStruct((NC, NR, D), jnp.float32),
      mesh=_vec_mesh,
      scratch_types=[
          pltpu.VMEM((3, 1, B), jnp.int32),
          pltpu.VMEM((3, 1, B), jnp.int32),
          pltpu.VMEM((3, 1, B), jnp.int32),
          pltpu.VMEM((B, D), jnp.float32),
          pltpu.VMEM((B, D), jnp.float32),
          pltpu.VMEM((B, D), jnp.float32),
          pltpu.VMEM_SHARED((NR, D), jnp.float32),
          pltpu.SemaphoreType.DMA,
          pltpu.SemaphoreType.DMA,
          pltpu.SemaphoreType.DMA,
          pltpu.SemaphoreType.DMA,
          pltpu.SemaphoreType.DMA,
          pltpu.SemaphoreType.DMA,
      ],
      compiler_params=_sc_params,
  )
  def seg_kernel(y_hbm, srcb_hbm, dstb_hbm, zero_hbm, out_hbm,
                 si, di, dr, rows0, rows1, rows2, acc,
                 gs0, gs1, gs2, ss0, ss1, ss2):
    cid = lax.axis_index("c")
    sid = lax.axis_index("s")
    base = cid * NHALF
    rows = (rows0, rows1, rows2)
    gsem = (gs0, gs1, gs2)
    ssem = (ss0, ss1, ss2)

    roff = pl.multiple_of(sid * CH, 8)
    pltpu.sync_copy(zero_hbm.at[pl.ds(roff, CH)], acc.at[pl.ds(roff, CH)])
    plsc.subcore_barrier()

    def load_idx(k, s):
      b = k * NS + sid
      pltpu.sync_copy(srcb_hbm.at[b], si.at[s])
      pltpu.sync_copy(dstb_hbm.at[b], di.at[s])

      @pl.loop(0, B, step=LANES)
      def _(j):
        d = di[s, 0, pl.ds(j, LANES)] - base
        ok = (d >= 0) & (d < NHALF)
        dr[s, 0, pl.ds(j, LANES)] = jnp.where(ok, d, DUMP)

    def g_start(s):
      pltpu.async_copy(y_hbm.at[si.at[s].at[0]], rows[s], gsem[s])

    def g_wait(s):
      pltpu.make_async_copy(y_hbm.at[si.at[s].at[0]], rows[s],
                            gsem[s]).wait()

    def s_start(s):
      pltpu.async_copy(rows[s], acc.at[dr.at[s].at[0]], ssem[s], add=True)

    def s_wait(s):
      pltpu.make_async_copy(rows[s], acc.at[dr.at[s].at[0]],
                            ssem[s]).wait()

    load_idx(0, 0)
    g_start(0)
    load_idx(1, 1)
    g_start(1)

    @pl.loop(0, GMAX + 2, step=3)
    def _(k):
      for j in range(3):
        b = k + j
        s = j
        s2 = (j + 2) % 3

        @pl.when(b < GMAX)
        def _():
          @pl.when(b + 2 < GMAX)
          def _():
            @pl.when(b >= 1)
            def _():
              s_wait(s2)

            load_idx(b + 2, s2)
            g_start(s2)

          g_wait(s)
          s_start(s)

    s_wait((GMAX - 1) % 3)
    s_wait((GMAX - 2) % 3)
    s_wait((GMAX - 3) % 3)

    plsc.subcore_barrier()
    pltpu.sync_copy(acc.at[pl.ds(roff, CH)],
                    out_hbm.at[cid].at[pl.ds(roff, CH)])

  zero = jnp.zeros((NR, D), jnp.float32)
  return seg_kernel(y, srcb3, dstb3, zero)


def _sc_segsum_scalar(z, src3, dst3):
  @pl.kernel(
      out_type=jax.ShapeDtypeStruct((NW, 1, N), jnp.float32),
      mesh=_vec_mesh,
      scratch_types=[
          pltpu.VMEM((1, N), jnp.float32),
          pltpu.VMEM((1, EPW), jnp.int32),
          pltpu.VMEM((1, EPW), jnp.int32),
          pltpu.VMEM((1, N), jnp.float32),
      ],
      compiler_params=_sc_params,
  )
  def seg2_kernel(z_hbm, src_hbm, dst_hbm, out_hbm, zv, sv, dv, acc):
    cid = lax.axis_index("c")
    sid = lax.axis_index("s")
    wid = sid * NC + cid

    pltpu.sync_copy(z_hbm, zv)
    pltpu.sync_copy(src_hbm.at[wid], sv)
    pltpu.sync_copy(dst_hbm.at[wid], dv)

    zeros = jnp.zeros((LANES,), jnp.float32)

    @pl.loop(0, N, step=LANES)
    def _(i):
      acc[0, pl.ds(i, LANES)] = zeros

    @pl.loop(0, EPW, step=LANES)
    def _(i):
      s = sv[0, pl.ds(i, LANES)]
      d = dv[0, pl.ds(i, LANES)]
      vals = plsc.load_gather(zv.at[0], [s])
      plsc.addupdate_scatter(acc.at[0], [d], vals)

    pltpu.sync_copy(acc, out_hbm.at[wid])

  return seg2_kernel(z, src3, dst3)


def _norm_col(partials_ref):
  return lax.rsqrt(
      jnp.clip(jnp.sum(partials_ref[...], axis=0), 1.0, None))[:, None]


def _tc_mm1(x, w1, doutp):
  def body(doutp_ref, x_ref, w1_ref, y_ref):
    no = _norm_col(doutp_ref)
    y_ref[...] = (
        jnp.dot(x_ref[...], w1_ref[...], preferred_element_type=jnp.float32)
        * no)

  return pl.pallas_call(
      body,
      in_specs=[
          pl.BlockSpec((NW, N), lambda: (0, 0)),
          pl.BlockSpec((N, D), lambda: (0, 0)),
          pl.BlockSpec((D, D), lambda: (0, 0)),
      ],
      out_specs=pl.BlockSpec((N, D), lambda: (0, 0)),
      out_shape=jax.ShapeDtypeStruct((N, D), jnp.float32),
  )(doutp, x, w1)


RB = 1000


def _tc_mm2(aggp, nin, nout, b1, w2t):
  bpc = NHALF // RB

  def body(agg_ref, ni_ref, no_ref, b1_ref, w2_ref, z_ref):
    h = jnp.maximum(agg_ref[0] * ni_ref[...] + b1_ref[...], 0.0)
    z_ref[...] = jnp.sum(h * w2_ref[...], axis=1, keepdims=True) * no_ref[...]

  return pl.pallas_call(
      body,
      grid=(N // RB,),
      in_specs=[
          pl.BlockSpec((1, RB, D), lambda i: (i // bpc, i % bpc, 0)),
          pl.BlockSpec((RB, 1), lambda i: (i, 0)),
          pl.BlockSpec((RB, 1), lambda i: (i, 0)),
          pl.BlockSpec((1, D), lambda i: (0, 0)),
          pl.BlockSpec((1, D), lambda i: (0, 0)),
      ],
      out_specs=pl.BlockSpec((RB, 1), lambda i: (i, 0)),
      out_shape=jax.ShapeDtypeStruct((N, 1), jnp.float32),
  )(aggp, nin, nout, b1, w2t)


def _tc_norms2(doutp, dinp):
  def body(doutp_ref, dinp_ref, no_ref, ni_ref, nir_ref):
    ni = lax.rsqrt(jnp.clip(jnp.sum(dinp_ref[...], axis=0), 1.0, None))
    nir_ref[...] = ni[None, :]
    ni_ref[...] = ni[:, None]
    no_ref[...] = _norm_col(doutp_ref)

  return pl.pallas_call(
      body,
      in_specs=[
          pl.BlockSpec((NW, N), lambda: (0, 0)),
          pl.BlockSpec((NW, N), lambda: (0, 0)),
      ],
      out_specs=[
          pl.BlockSpec((N, 1), lambda: (0, 0)),
          pl.BlockSpec((N, 1), lambda: (0, 0)),
          pl.BlockSpec((1, N), lambda: (0, 0)),
      ],
      out_shape=[
          jax.ShapeDtypeStruct((N, 1), jnp.float32),
          jax.ShapeDtypeStruct((N, 1), jnp.float32),
          jax.ShapeDtypeStruct((1, N), jnp.float32),
      ],
  )(doutp, dinp)


def _tc_out(a2p, nin_row, b2):
  def body(a2p_ref, ni_ref, b2_ref, o_ref):
    s = jnp.sum(a2p_ref[...], axis=0, keepdims=True)
    o_ref[...] = jax.nn.sigmoid(s * ni_ref[...] + b2_ref[0, 0])

  return pl.pallas_call(
      body,
      in_specs=[
          pl.BlockSpec((NW, N), lambda: (0, 0)),
          pl.BlockSpec((1, N), lambda: (0, 0)),
          pl.BlockSpec((1, 1), lambda: (0, 0)),
      ],
      out_specs=pl.BlockSpec((1, N), lambda: (0, 0)),
      out_shape=jax.ShapeDtypeStruct((1, N), jnp.float32),
  )(a2p, nin_row, b2)


def kernel(x, edge_index, W1, b1, W2, b2):
  src = edge_index[0].astype(jnp.int32)
  dst = edge_index[1].astype(jnp.int32)
  srcb3 = src.reshape(NB, 1, B)
  dstb3 = dst.reshape(NB, 1, B)
  src3 = src.reshape(NW, 1, EPW)
  dst3 = dst.reshape(NW, 1, EPW)

  doutp, dinp = _sc_degrees(src3, dst3)
  doutp = doutp.reshape(NW, N)
  dinp = dinp.reshape(NW, N)
  y = _tc_mm1(x, W1, doutp)
  nout, nin, nin_row = _tc_norms2(doutp, dinp)
  aggp = _sc_segsum_rows(y, srcb3, dstb3)
  z = _tc_mm2(aggp, nin, nout, b1.reshape(1, D), W2.reshape(1, D))
  a2p = _sc_segsum_scalar(z.reshape(1, N), src3, dst3)
  out = _tc_out(a2p.reshape(NW, N), nin_row, b2.reshape(1, 1))
  return out.reshape(N, 1)


# speedup vs baseline: 1.5638x; 1.0011x over previous
import jax
import jax.numpy as jnp
from jax import lax
from jax.experimental import pallas as pl
from jax.experimental.pallas import tpu as pltpu
from jax.experimental.pallas import tpu_sc as plsc

N = 10000
E = 320000
D = 128

NC = 2
NS = 16
NW = NC * NS
LANES = 16

EPW = E // NW
B = 200
NB = E // B
GMAX = NB // NS
NHALF = 5000
NR = 5120
DUMP = NHALF
CH = NR // NS

_vec_mesh = plsc.VectorSubcoreMesh(
    core_axis_name="c", subcore_axis_name="s", num_cores=NC, num_subcores=NS
)

_sc_params = pltpu.CompilerParams(needs_layout_passes=False)


def _sc_degrees(src3, dst3):
  @pl.kernel(
      out_type=(
          jax.ShapeDtypeStruct((NW, 1, N), jnp.float32),
          jax.ShapeDtypeStruct((NW, 1, N), jnp.float32),
      ),
      mesh=_vec_mesh,
      scratch_types=[
          pltpu.VMEM((1, EPW), jnp.int32),
          pltpu.VMEM((1, EPW), jnp.int32),
          pltpu.VMEM((1, N), jnp.float32),
          pltpu.VMEM((1, N), jnp.float32),
      ],
      compiler_params=_sc_params,
  )
  def deg_kernel(src_hbm, dst_hbm, dout_hbm, din_hbm, sv, dv, aout, ain):
    cid = lax.axis_index("c")
    sid = lax.axis_index("s")
    wid = sid * NC + cid

    pltpu.sync_copy(src_hbm.at[wid], sv)
    pltpu.sync_copy(dst_hbm.at[wid], dv)

    zeros = jnp.zeros((LANES,), jnp.float32)
    ones = jnp.ones((LANES,), jnp.float32)

    @pl.loop(0, N, step=LANES)
    def _(i):
      aout[0, pl.ds(i, LANES)] = zeros
      ain[0, pl.ds(i, LANES)] = zeros

    @pl.loop(0, EPW, step=LANES)
    def _(i):
      s = sv[0, pl.ds(i, LANES)]
      d = dv[0, pl.ds(i, LANES)]
      plsc.addupdate_scatter(aout.at[0], [s], ones)
      plsc.addupdate_scatter(ain.at[0], [d], ones)

    pltpu.sync_copy(aout, dout_hbm.at[wid])
    pltpu.sync_copy(ain, din_hbm.at[wid])

  return deg_kernel(src3, dst3)


def _sc_segsum_rows(y, srcb3, dstb3):
  @pl.kernel(
      out_type=jax.ShapeDtypeStruct((NC, NR, D), jnp.float32),
      mesh=_vec_mesh,
      scratch_types=[
          pltpu.VMEM((3, 1, B), jnp.int32),
          pltpu.VMEM((3, 1, B), jnp.int32),
          pltpu.VMEM((3, 1, B), jnp.int32),
          pltpu.VMEM((B, D), jnp.float32),
          pltpu.VMEM((B, D), jnp.float32),
          pltpu.VMEM((B, D), jnp.float32),
          pltpu.VMEM_SHARED((NR, D), jnp.float32),
          pltpu.SemaphoreType.DMA,
          pltpu.SemaphoreType.DMA,
          pltpu.SemaphoreType.DMA,
          pltpu.SemaphoreType.DMA,
          pltpu.SemaphoreType.DMA,
          pltpu.SemaphoreType.DMA,
      ],
      compiler_params=_sc_params,
  )
  def seg_kernel(y_hbm, srcb_hbm, dstb_hbm, zero_hbm, out_hbm,
                 si, di, dr, rows0, rows1, rows2, acc,
                 gs0, gs1, gs2, ss0, ss1, ss2):
    cid = lax.axis_index("c")
    sid = lax.axis_index("s")
    base = cid * NHALF
    rows = (rows0, rows1, rows2)
    gsem = (gs0, gs1, gs2)
    ssem = (ss0, ss1, ss2)

    roff = pl.multiple_of(sid * CH, 8)
    pltpu.sync_copy(zero_hbm.at[pl.ds(roff, CH)], acc.at[pl.ds(roff, CH)])
    plsc.subcore_barrier()

    def load_idx(k, s):
      b = k * NS + sid
      pltpu.sync_copy(srcb_hbm.at[b], si.at[s])
      pltpu.sync_copy(dstb_hbm.at[b], di.at[s])

      @pl.loop(0, B, step=LANES)
      def _(j):
        d = di[s, 0, pl.ds(j, LANES)] - base
        ok = (d >= 0) & (d < NHALF)
        dr[s, 0, pl.ds(j, LANES)] = jnp.where(ok, d, DUMP)

    def g_start(s):
      pltpu.async_copy(y_hbm.at[si.at[s].at[0]], rows[s], gsem[s])

    def g_wait(s):
      pltpu.make_async_copy(y_hbm.at[si.at[s].at[0]], rows[s],
                            gsem[s]).wait()

    def s_start(s):
      pltpu.async_copy(rows[s], acc.at[dr.at[s].at[0]], ssem[s], add=True)

    def s_wait(s):
      pltpu.make_async_copy(rows[s], acc.at[dr.at[s].at[0]],
                            ssem[s]).wait()

    load_idx(0, 0)
    g_start(0)
    load_idx(1, 1)
    g_start(1)

    @pl.loop(0, GMAX + 2, step=3)
    def _(k):
      for j in range(3):
        b = k + j
        s = j
        s2 = (j + 2) % 3

        @pl.when(b < GMAX)
        def _():
          @pl.when(b + 2 < GMAX)
          def _():
            @pl.when(b >= 1)
            def _():
              s_wait(s2)

            load_idx(b + 2, s2)
            g_start(s2)

          g_wait(s)
          s_start(s)

    s_wait((GMAX - 1) % 3)
    s_wait((GMAX - 2) % 3)
    s_wait((GMAX - 3) % 3)

    plsc.subcore_barrier()
    pltpu.sync_copy(acc.at[pl.ds(roff, CH)],
                    out_hbm.at[cid].at[pl.ds(roff, CH)])

  zero = jnp.zeros((NR, D), jnp.float32)
  return seg_kernel(y, srcb3, dstb3, zero)


def _sc_segsum_scalar(z, src3, dst3):
  @pl.kernel(
      out_type=jax.ShapeDtypeStruct((NW, 1, N), jnp.float32),
      mesh=_vec_mesh,
      scratch_types=[
          pltpu.VMEM((1, N), jnp.float32),
          pltpu.VMEM((1, EPW), jnp.int32),
          pltpu.VMEM((1, EPW), jnp.int32),
          pltpu.VMEM((1, N), jnp.float32),
      ],
      compiler_params=_sc_params,
  )
  def seg2_kernel(z_hbm, src_hbm, dst_hbm, out_hbm, zv, sv, dv, acc):
    cid = lax.axis_index("c")
    sid = lax.axis_index("s")
    wid = sid * NC + cid

    pltpu.sync_copy(z_hbm, zv)
    pltpu.sync_copy(src_hbm.at[wid], sv)
    pltpu.sync_copy(dst_hbm.at[wid], dv)

    zeros = jnp.zeros((LANES,), jnp.float32)

    @pl.loop(0, N, step=LANES)
    def _(i):
      acc[0, pl.ds(i, LANES)] = zeros

    @pl.loop(0, EPW, step=LANES)
    def _(i):
      s = sv[0, pl.ds(i, LANES)]
      d = dv[0, pl.ds(i, LANES)]
      vals = plsc.load_gather(zv.at[0], [s])
      plsc.addupdate_scatter(acc.at[0], [d], vals)

    pltpu.sync_copy(acc, out_hbm.at[wid])

  return seg2_kernel(z, src3, dst3)


def _norm_col(partials_ref):
  return lax.rsqrt(
      jnp.clip(jnp.sum(partials_ref[...], axis=0), 1.0, None))[:, None]


def _tc_mm1(x, w1, doutp):
  def body(doutp_ref, x_ref, w1_ref, y_ref):
    no = _norm_col(doutp_ref)
    y_ref[...] = (
        jnp.dot(x_ref[...], w1_ref[...], preferred_element_type=jnp.float32)
        * no)

  return pl.pallas_call(
      body,
      in_specs=[
          pl.BlockSpec((NW, N), lambda: (0, 0)),
          pl.BlockSpec((N, D), lambda: (0, 0)),
          pl.BlockSpec((D, D), lambda: (0, 0)),
      ],
      out_specs=pl.BlockSpec((N, D), lambda: (0, 0)),
      out_shape=jax.ShapeDtypeStruct((N, D), jnp.float32),
  )(doutp, x, w1)


RB = 1000


def _tc_mm2(aggp, nin, nout, b1, w2t):
  bpc = NHALF // RB

  def body(agg_ref, ni_ref, no_ref, b1_ref, w2_ref, z_ref):
    h = jnp.maximum(agg_ref[0] * ni_ref[...] + b1_ref[...], 0.0)
    z_ref[...] = jnp.sum(h * w2_ref[...], axis=1, keepdims=True) * no_ref[...]

  return pl.pallas_call(
      body,
      grid=(N // RB,),
      in_specs=[
          pl.BlockSpec((1, RB, D), lambda i: (i // bpc, i % bpc, 0)),
          pl.BlockSpec((RB, 1), lambda i: (i, 0)),
          pl.BlockSpec((RB, 1), lambda i: (i, 0)),
          pl.BlockSpec((1, D), lambda i: (0, 0)),
          pl.BlockSpec((1, D), lambda i: (0, 0)),
      ],
      out_specs=pl.BlockSpec((RB, 1), lambda i: (i, 0)),
      out_shape=jax.ShapeDtypeStruct((N, 1), jnp.float32),
  )(aggp, nin, nout, b1, w2t)


def _tc_norms2(doutp, dinp):
  def body(doutp_ref, dinp_ref, no_ref, ni_ref, nir_ref):
    ni = lax.rsqrt(jnp.clip(jnp.sum(dinp_ref[...], axis=0), 1.0, None))
    nir_ref[...] = ni[None, :]
    ni_ref[...] = ni[:, None]
    no_ref[...] = _norm_col(doutp_ref)

  return pl.pallas_call(
      body,
      in_specs=[
          pl.BlockSpec((NW, N), lambda: (0, 0)),
          pl.BlockSpec((NW, N), lambda: (0, 0)),
      ],
      out_specs=[
          pl.BlockSpec((N, 1), lambda: (0, 0)),
          pl.BlockSpec((N, 1), lambda: (0, 0)),
          pl.BlockSpec((1, N), lambda: (0, 0)),
      ],
      out_shape=[
          jax.ShapeDtypeStruct((N, 1), jnp.float32),
          jax.ShapeDtypeStruct((N, 1), jnp.float32),
          jax.ShapeDtypeStruct((1, N), jnp.float32),
      ],
  )(doutp, dinp)


def _tc_out(a2p, nin_row, b2):
  def body(a2p_ref, ni_ref, b2_ref, o_ref):
    s = jnp.sum(a2p_ref[...], axis=0, keepdims=True)
    o_ref[...] = jax.nn.sigmoid(s * ni_ref[...] + b2_ref[0, 0])

  return pl.pallas_call(
      body,
      in_specs=[
          pl.BlockSpec((NW, N), lambda: (0, 0)),
          pl.BlockSpec((1, N), lambda: (0, 0)),
          pl.BlockSpec((1, 1), lambda: (0, 0)),
      ],
      out_specs=pl.BlockSpec((1, N), lambda: (0, 0)),
      out_shape=jax.ShapeDtypeStruct((1, N), jnp.float32),
  )(a2p, nin_row, b2)


def kernel(x, edge_index, W1, b1, W2, b2):
  src = edge_index[0].astype(jnp.int32)
  dst = edge_index[1].astype(jnp.int32)
  srcb3 = src.reshape(NB, 1, B)
  dstb3 = dst.reshape(NB, 1, B)
  src3 = src.reshape(NW, 1, EPW)
  dst3 = dst.reshape(NW, 1, EPW)

  doutp, dinp = _sc_degrees(src3, dst3)
  doutp = doutp.reshape(NW, N)
  dinp = dinp.reshape(NW, N)
  y = _tc_mm1(x, W1, doutp)
  nout, nin, nin_row = _tc_norms2(doutp, dinp)
  aggp = _sc_segsum_rows(y, srcb3, dstb3)
  z = _tc_mm2(aggp, nin, nout, b1.reshape(1, D), W2.reshape(1, D))
  a2p = _sc_segsum_scalar(z.reshape(1, N), src3, dst3)
  out = _tc_out(a2p.reshape(NW, N), nin_row, b2.reshape(1, 1))
  return out.reshape(N, 1)
